# baseline (speedup 1.0000x reference)
import functools
import jax
import jax.numpy as jnp
from jax import lax
from jax.experimental import pallas as pl
from jax.experimental.pallas import tpu as pltpu
from jax.experimental.pallas import tpu_sc as plsc

D = 64
SCALE = 8.0
NC, NS = 2, 16
NW = NC * NS
B = 4096 * 200
BPW = B // NW
IDX_MINOR = 128
NIDXROW = BPW // IDX_MINOR
CHUNK = 512
GPC = CHUNK // IDX_MINOR
NCHUNK = BPW // CHUNK

_mesh = plsc.VectorSubcoreMesh(
    core_axis_name="c", subcore_axis_name="s", num_cores=NC, num_subcores=NS
)


@functools.partial(
    pl.kernel,
    out_type=jax.ShapeDtypeStruct((B, D), jnp.float32),
    mesh=_mesh,
    scratch_types=[
        pltpu.VMEM((NIDXROW, IDX_MINOR), jnp.int32),
        pltpu.VMEM((CHUNK, D), jnp.float32),
        pltpu.SemaphoreType.DMA,
    ],
    compiler_params=pltpu.CompilerParams(use_tc_tiling_on_sc=False),
)
def _emb_lookup(x_hbm, lut_hbm, out_hbm, idx_v, rows_v, sem):
    wid = lax.axis_index("s") * NC + lax.axis_index("c")
    base = wid * BPW

    pltpu.sync_copy(x_hbm.at[wid], idx_v)

    @pl.loop(0, NCHUNK)
    def _chunk(g):
        copies = [
            pltpu.async_copy(
                lut_hbm.at[idx_v.at[g * GPC + j]],
                rows_v.at[pl.ds(j * IDX_MINOR, IDX_MINOR)],
                sem,
            )
            for j in range(GPC)
        ]
        for cp in copies:
            cp.wait()

        @pl.loop(0, CHUNK, unroll=8)
        def _scale(r):
            for c in range(D // 16):
                rows_v[r, pl.ds(c * 16, 16)] = (
                    rows_v[r, pl.ds(c * 16, 16)] * SCALE
                )

        pltpu.sync_copy(rows_v, out_hbm.at[pl.ds(base + g * CHUNK, CHUNK)])


def kernel(x, lut):
    xf = x.reshape(NW, NIDXROW, IDX_MINOR).astype(jnp.int32)
    out = _emb_lookup(xf, lut)
    return out.reshape(x.shape[0], x.shape[1], D)


# speedup vs baseline: 1.0717x; 1.0717x over previous
import functools
import jax
import jax.numpy as jnp
from jax import lax
from jax.experimental import pallas as pl
from jax.experimental.pallas import tpu as pltpu
from jax.experimental.pallas import tpu_sc as plsc

D = 64
SCALE = 8.0
NC, NS = 2, 16
NW = NC * NS
B = 4096 * 200
BPW = B // NW
IDX_MINOR = 128
NIDXROW = BPW // IDX_MINOR
CHUNK = 256
GPC = CHUNK // IDX_MINOR
NCHUNK = BPW // CHUNK
NBUF = 4

_mesh = plsc.VectorSubcoreMesh(
    core_axis_name="c", subcore_axis_name="s", num_cores=NC, num_subcores=NS
)


@functools.partial(
    pl.kernel,
    out_type=jax.ShapeDtypeStruct((B, D), jnp.float32),
    mesh=_mesh,
    scratch_types=[
        pltpu.VMEM((NIDXROW, IDX_MINOR), jnp.int32),
        [pltpu.VMEM((CHUNK, D), jnp.float32) for _ in range(NBUF)],
        [pltpu.SemaphoreType.DMA for _ in range(NBUF)],
        [pltpu.SemaphoreType.DMA for _ in range(NBUF)],
    ],
    compiler_params=pltpu.CompilerParams(use_tc_tiling_on_sc=False),
)
def _emb_lookup(x_hbm, lut_hbm, out_hbm, idx_v, rows, gsem, osem):
    wid = lax.axis_index("s") * NC + lax.axis_index("c")
    base = wid * BPW

    pltpu.sync_copy(x_hbm.at[wid], idx_v)

    def fire_gathers(g, r):
        for j in range(GPC):
            pltpu.async_copy(
                lut_hbm.at[idx_v.at[g * GPC + j]],
                rows[r].at[pl.ds(j * IDX_MINOR, IDX_MINOR)],
                gsem[r],
            )

    def drain_gathers(g, r):
        for j in range(GPC):
            pltpu.make_async_copy(
                lut_hbm.at[idx_v.at[g * GPC + j]],
                rows[r].at[pl.ds(j * IDX_MINOR, IDX_MINOR)],
                gsem[r],
            ).wait()

    def scale(r):
        @pl.loop(0, CHUNK, unroll=8)
        def _scale(row):
            for c in range(D // 16):
                rows[r][row, pl.ds(c * 16, 16)] = (
                    rows[r][row, pl.ds(c * 16, 16)] * SCALE
                )

    def fire_write(g, r):
        pltpu.async_copy(
            rows[r], out_hbm.at[pl.ds(base + g * CHUNK, CHUNK)], osem[r]
        )

    def drain_write(g, r):
        pltpu.make_async_copy(
            rows[r], out_hbm.at[pl.ds(base + g * CHUNK, CHUNK)], osem[r]
        ).wait()

    for r in range(NBUF - 1):
        fire_gathers(r, r)

    @pl.loop(0, NCHUNK // NBUF)
    def _step(k):
        for r in range(NBUF):
            g = k * NBUF + r
            drain_gathers(g, r)
            scale(r)

            rn = (r + NBUF - 1) % NBUF
            gn = g + NBUF - 1

            @pl.when(gn < NCHUNK)
            def _():
                @pl.when(g >= 1)
                def _():
                    drain_write(gn - NBUF, rn)

                fire_gathers(gn, rn)

            fire_write(g, r)

    for r in range(NBUF):
        g = NCHUNK - NBUF + r
        drain_write(g, g % NBUF)


def kernel(x, lut):
    xf = x.reshape(NW, NIDXROW, IDX_MINOR).astype(jnp.int32)
    out = _emb_lookup(xf, lut)
    return out.reshape(x.shape[0], x.shape[1], D)
